# baseline (speedup 1.0000x reference)
import functools

import jax
import jax.numpy as jnp
from jax import lax
from jax.experimental import pallas as pl
from jax.experimental.pallas import tpu as pltpu
from jax.experimental.pallas import tpu_sc as plsc

B, D, H, W = 8, 64, 32, 32
N = B * H * W
K = 16384
CP = 128

NT = 128
NN = N // NT


def _dist_body(w_ref, wT_ref, tokT_ref, bc_ref, br_ref, zf_ref,
               d_ref, idx_ref, cbp_ref, cbT2_ref, cbsq_ref):
    n = pl.program_id(0)

    @pl.when(n == 0)
    def _():
        cbT = (
            jnp.dot(w_ref[...], tokT_ref[...],
                    preferred_element_type=jnp.float32)
            + bc_ref[...]
        )
        cbT2_ref[...] = cbT + cbT
        cbsq_ref[...] = jnp.sum(cbT * cbT, axis=0, keepdims=True)
        cbp_ref[:, :D] = (
            lax.dot_general(tokT_ref[...], wT_ref[...],
                            (((0,), (0,)), ((), ())),
                            preferred_element_type=jnp.float32)
            + br_ref[...]
        )

    zf = zf_ref[...]
    mm2 = jnp.dot(zf, cbT2_ref[...], preferred_element_type=jnp.float32)
    zsq = jnp.sum(zf * zf, axis=1, keepdims=True)
    d = (zsq + cbsq_ref[...]) - mm2
    d_ref[...] = d
    tmin = jnp.min(d, axis=1, keepdims=True)
    iota = lax.broadcasted_iota(jnp.int32, (NT, K), 1)
    idx_ref[...] = jnp.min(jnp.where(d == tmin, iota, K), axis=1)


_NC, _NS = 2, 16
NWORK = _NC * _NS
RPW = N // NWORK


def _gather_st_body(cb_ref, idx_ref, zf_ref, zq_ref, part_ref,
                    idx_v, rows_v, z_v, o_v, acc_v, sem, zsem):
    wid = lax.axis_index("s") * _NC + lax.axis_index("c")
    base = wid * RPW
    zcp = pltpu.async_copy(zf_ref.at[pl.ds(base, RPW)], z_v, zsem)
    pltpu.sync_copy(idx_ref.at[pl.ds(base, RPW)], idx_v)
    pltpu.async_copy(cb_ref.at[idx_v], rows_v, sem).wait()
    zcp.wait()

    zero = jnp.zeros((16,), jnp.float32)
    HALF = RPW // 2

    @plsc.parallel_loop(0, HALF, unroll=4, carry=(zero, zero, zero, zero))
    def acc_lo(r, a):
        out = []
        for c in range(D // 16):
            q = rows_v[r, pl.ds(c * 16, 16)]
            zz = z_v[r, pl.ds(c * 16, 16)]
            dq = q - zz
            o_v[r, pl.ds(c * 16, 16)] = zz + dq
            out.append(a[c] + dq * dq)
        return tuple(out)

    wb = pltpu.async_copy(o_v.at[pl.ds(0, HALF)],
                          zq_ref.at[pl.ds(base, HALF)], zsem)

    @plsc.parallel_loop(HALF, RPW, unroll=4, carry=acc_lo)
    def accs(r, a):
        out = []
        for c in range(D // 16):
            q = rows_v[r, pl.ds(c * 16, 16)]
            zz = z_v[r, pl.ds(c * 16, 16)]
            dq = q - zz
            o_v[r, pl.ds(c * 16, 16)] = zz + dq
            out.append(a[c] + dq * dq)
        return tuple(out)

    acc_v[...] = (accs[0] + accs[1]) + (accs[2] + accs[3])
    pltpu.sync_copy(o_v.at[pl.ds(HALF, HALF)],
                    zq_ref.at[pl.ds(base + HALF, HALF)])
    pltpu.sync_copy(acc_v, part_ref.at[pl.ds(wid * 16, 16)])
    wb.wait()


def kernel(z, tok_embeddings, proj_w, proj_b):
    zp = jnp.transpose(z, (0, 2, 3, 1))
    zf = zp.reshape(N, D)

    d, idx, cb_pad = pl.pallas_call(
        _dist_body,
        grid=(NN,),
        in_specs=[
            pl.BlockSpec((D, D), lambda n: (0, 0)),
            pl.BlockSpec((D, D), lambda n: (0, 0)),
            pl.BlockSpec((D, K), lambda n: (0, 0)),
            pl.BlockSpec((D, 1), lambda n: (0, 0)),
            pl.BlockSpec((1, D), lambda n: (0, 0)),
            pl.BlockSpec((NT, D), lambda n: (n, 0)),
        ],
        out_specs=[
            pl.BlockSpec((NT, K), lambda n: (n, 0)),
            pl.BlockSpec((NT,), lambda n: (n,)),
            pl.BlockSpec((K, CP), lambda n: (0, 0)),
        ],
        out_shape=[
            jax.ShapeDtypeStruct((N, K), jnp.float32),
            jax.ShapeDtypeStruct((N,), jnp.int32),
            jax.ShapeDtypeStruct((K, CP), jnp.float32),
        ],
        scratch_shapes=[
            pltpu.VMEM((D, K), jnp.float32),
            pltpu.VMEM((1, K), jnp.float32),
        ],
        compiler_params=pltpu.CompilerParams(
            dimension_semantics=("arbitrary",),
        ),
    )(proj_w, proj_w.T, tok_embeddings.T,
      proj_b.reshape(D, 1), proj_b.reshape(1, D), zf)

    mesh = plsc.VectorSubcoreMesh(core_axis_name="c", subcore_axis_name="s")
    zq_st, partials = pl.kernel(
        _gather_st_body,
        mesh=mesh,
        out_type=[
            jax.ShapeDtypeStruct((N, D), jnp.float32),
            jax.ShapeDtypeStruct((NWORK * 16,), jnp.float32),
        ],
        scratch_types=[
            pltpu.VMEM((RPW,), jnp.int32),
            pltpu.VMEM((RPW, CP), jnp.float32),
            pltpu.VMEM((RPW, D), jnp.float32),
            pltpu.VMEM((RPW, D), jnp.float32),
            pltpu.VMEM((16,), jnp.float32),
            pltpu.SemaphoreType.DMA,
            pltpu.SemaphoreType.DMA,
        ],
    )(cb_pad, idx, zf)

    m = jnp.sum(partials) / (N * D)
    loss = m + 0.33 * m
    z_q_out = jnp.transpose(zq_st.reshape(B, H, W, D), (0, 3, 1, 2))
    return (z_q_out, loss, d, idx)


# speedup vs baseline: 1.0655x; 1.0655x over previous
import functools

import jax
import jax.numpy as jnp
from jax import lax
from jax.experimental import pallas as pl
from jax.experimental.pallas import tpu as pltpu
from jax.experimental.pallas import tpu_sc as plsc

B, D, H, W = 8, 64, 32, 32
N = B * H * W
K = 16384
CP = 128

NT = 256
NN = N // NT


def _dist_body(w_ref, wT_ref, tokT_ref, bc_ref, br_ref, zf_ref,
               d_ref, idx_ref, cbp_ref, cbT2_ref, cbsq_ref):
    n = pl.program_id(0)

    @pl.when(n == 0)
    def _():
        cbT = (
            jnp.dot(w_ref[...], tokT_ref[...],
                    preferred_element_type=jnp.float32)
            + bc_ref[...]
        )
        cbT2_ref[...] = cbT + cbT
        cbsq_ref[...] = jnp.sum(cbT * cbT, axis=0, keepdims=True)
        cbp_ref[:, :D] = (
            lax.dot_general(tokT_ref[...], wT_ref[...],
                            (((0,), (0,)), ((), ())),
                            preferred_element_type=jnp.float32)
            + br_ref[...]
        )

    zf = zf_ref[...]
    mm2 = jnp.dot(zf, cbT2_ref[...], preferred_element_type=jnp.float32)
    zsq = jnp.sum(zf * zf, axis=1, keepdims=True)
    d = (zsq + cbsq_ref[...]) - mm2
    d_ref[...] = d
    tmin = jnp.min(d, axis=1, keepdims=True)
    iota = lax.broadcasted_iota(jnp.int32, (NT, K), 1)
    idx_ref[...] = jnp.min(jnp.where(d == tmin, iota, K), axis=1)


_NC, _NS = 2, 16
NWORK = _NC * _NS
RPW = N // NWORK


def _gather_st_body(cb_ref, idx_ref, zf_ref, zq_ref, part_ref,
                    idx_v, rows_v, z_v, o_v, acc_v, sem, zsem):
    wid = lax.axis_index("s") * _NC + lax.axis_index("c")
    base = wid * RPW
    zcp = pltpu.async_copy(zf_ref.at[pl.ds(base, RPW)], z_v, zsem)
    pltpu.sync_copy(idx_ref.at[pl.ds(base, RPW)], idx_v)
    pltpu.async_copy(cb_ref.at[idx_v], rows_v, sem).wait()
    zcp.wait()

    zero = jnp.zeros((16,), jnp.float32)
    HALF = RPW // 2

    @plsc.parallel_loop(0, HALF, unroll=4, carry=(zero, zero, zero, zero))
    def acc_lo(r, a):
        out = []
        for c in range(D // 16):
            q = rows_v[r, pl.ds(c * 16, 16)]
            zz = z_v[r, pl.ds(c * 16, 16)]
            dq = q - zz
            o_v[r, pl.ds(c * 16, 16)] = zz + dq
            out.append(a[c] + dq * dq)
        return tuple(out)

    wb = pltpu.async_copy(o_v.at[pl.ds(0, HALF)],
                          zq_ref.at[pl.ds(base, HALF)], zsem)

    @plsc.parallel_loop(HALF, RPW, unroll=4, carry=acc_lo)
    def accs(r, a):
        out = []
        for c in range(D // 16):
            q = rows_v[r, pl.ds(c * 16, 16)]
            zz = z_v[r, pl.ds(c * 16, 16)]
            dq = q - zz
            o_v[r, pl.ds(c * 16, 16)] = zz + dq
            out.append(a[c] + dq * dq)
        return tuple(out)

    acc_v[...] = (accs[0] + accs[1]) + (accs[2] + accs[3])
    pltpu.sync_copy(o_v.at[pl.ds(HALF, HALF)],
                    zq_ref.at[pl.ds(base + HALF, HALF)])
    pltpu.sync_copy(acc_v, part_ref.at[pl.ds(wid * 16, 16)])
    wb.wait()


def kernel(z, tok_embeddings, proj_w, proj_b):
    zp = jnp.transpose(z, (0, 2, 3, 1))
    zf = zp.reshape(N, D)

    d, idx, cb_pad = pl.pallas_call(
        _dist_body,
        grid=(NN,),
        in_specs=[
            pl.BlockSpec((D, D), lambda n: (0, 0)),
            pl.BlockSpec((D, D), lambda n: (0, 0)),
            pl.BlockSpec((D, K), lambda n: (0, 0)),
            pl.BlockSpec((D, 1), lambda n: (0, 0)),
            pl.BlockSpec((1, D), lambda n: (0, 0)),
            pl.BlockSpec((NT, D), lambda n: (n, 0)),
        ],
        out_specs=[
            pl.BlockSpec((NT, K), lambda n: (n, 0)),
            pl.BlockSpec((NT,), lambda n: (n,)),
            pl.BlockSpec((K, CP), lambda n: (0, 0)),
        ],
        out_shape=[
            jax.ShapeDtypeStruct((N, K), jnp.float32),
            jax.ShapeDtypeStruct((N,), jnp.int32),
            jax.ShapeDtypeStruct((K, CP), jnp.float32),
        ],
        scratch_shapes=[
            pltpu.VMEM((D, K), jnp.float32),
            pltpu.VMEM((1, K), jnp.float32),
        ],
        compiler_params=pltpu.CompilerParams(
            dimension_semantics=("arbitrary",),
        ),
    )(proj_w, proj_w.T, tok_embeddings.T,
      proj_b.reshape(D, 1), proj_b.reshape(1, D), zf)

    mesh = plsc.VectorSubcoreMesh(core_axis_name="c", subcore_axis_name="s")
    zq_st, partials = pl.kernel(
        _gather_st_body,
        mesh=mesh,
        out_type=[
            jax.ShapeDtypeStruct((N, D), jnp.float32),
            jax.ShapeDtypeStruct((NWORK * 16,), jnp.float32),
        ],
        scratch_types=[
            pltpu.VMEM((RPW,), jnp.int32),
            pltpu.VMEM((RPW, CP), jnp.float32),
            pltpu.VMEM((RPW, D), jnp.float32),
            pltpu.VMEM((RPW, D), jnp.float32),
            pltpu.VMEM((16,), jnp.float32),
            pltpu.SemaphoreType.DMA,
            pltpu.SemaphoreType.DMA,
        ],
    )(cb_pad, idx, zf)

    m = jnp.sum(partials) / (N * D)
    loss = m + 0.33 * m
    z_q_out = jnp.transpose(zq_st.reshape(B, H, W, D), (0, 3, 1, 2))
    return (z_q_out, loss, d, idx)


# speedup vs baseline: 1.0662x; 1.0007x over previous
import jax
import jax.numpy as jnp
from jax import lax
from jax.experimental import pallas as pl
from jax.experimental.pallas import tpu as pltpu
from jax.experimental.pallas import tpu_sc as plsc

B, D, H, W = 8, 64, 32, 32
N = B * H * W
K = 16384
CP = 128

NT = 256
NN = N // NT


def _dist_body(w_ref, wT_ref, tokT_ref, bc_ref, br_ref, zf_ref,
               d_ref, idx_ref, cbp_ref, cbT2_ref, cbsq_ref):
    n = pl.program_id(0)

    @pl.when(n == 0)
    def _():
        cbT = (
            jnp.dot(w_ref[...], tokT_ref[...],
                    preferred_element_type=jnp.float32)
            + bc_ref[...]
        )
        cbT2_ref[...] = cbT + cbT
        cbsq_ref[...] = jnp.sum(cbT * cbT, axis=0, keepdims=True)
        cbp_ref[:, :D] = (
            lax.dot_general(tokT_ref[...], wT_ref[...],
                            (((0,), (0,)), ((), ())),
                            preferred_element_type=jnp.float32)
            + br_ref[...]
        )

    zf = zf_ref[...]
    mm2 = jnp.dot(zf, cbT2_ref[...], preferred_element_type=jnp.float32)
    zsq = jnp.sum(zf * zf, axis=1, keepdims=True)
    d = (zsq + cbsq_ref[...]) - mm2
    d_ref[...] = d
    tmin = jnp.min(d, axis=1, keepdims=True)
    iota = lax.broadcasted_iota(jnp.int32, (NT, K), 1)
    idx_ref[...] = jnp.min(jnp.where(d == tmin, iota, K), axis=1)


_NC, _NS = 2, 16
NWORK = _NC * _NS
RPW = N // NWORK


def _gather_st_body(cb_ref, idx_ref, zf_ref, zq_ref, part_ref,
                    idx_v, rows_v, z_v, o_v, acc_v, sem, zsem):
    wid = lax.axis_index("s") * _NC + lax.axis_index("c")
    base = wid * RPW
    zcp = pltpu.async_copy(zf_ref.at[pl.ds(base, RPW)], z_v, zsem)
    pltpu.sync_copy(idx_ref.at[pl.ds(base, RPW)], idx_v)
    pltpu.async_copy(cb_ref.at[idx_v], rows_v, sem).wait()
    zcp.wait()

    zero = jnp.zeros((16,), jnp.float32)
    HALF = RPW // 2

    @plsc.parallel_loop(0, HALF, unroll=4, carry=(zero, zero, zero, zero))
    def acc_lo(r, a):
        out = []
        for c in range(D // 16):
            q = rows_v[r, pl.ds(c * 16, 16)]
            zz = z_v[r, pl.ds(c * 16, 16)]
            dq = q - zz
            o_v[r, pl.ds(c * 16, 16)] = zz + dq
            out.append(a[c] + dq * dq)
        return tuple(out)

    wb = pltpu.async_copy(o_v.at[pl.ds(0, HALF)],
                          zq_ref.at[pl.ds(base, HALF)], zsem)

    @plsc.parallel_loop(HALF, RPW, unroll=4, carry=acc_lo)
    def accs(r, a):
        out = []
        for c in range(D // 16):
            q = rows_v[r, pl.ds(c * 16, 16)]
            zz = z_v[r, pl.ds(c * 16, 16)]
            dq = q - zz
            o_v[r, pl.ds(c * 16, 16)] = zz + dq
            out.append(a[c] + dq * dq)
        return tuple(out)

    acc_v[...] = (accs[0] + accs[1]) + (accs[2] + accs[3])
    pltpu.sync_copy(o_v.at[pl.ds(HALF, HALF)],
                    zq_ref.at[pl.ds(base + HALF, HALF)])
    pltpu.sync_copy(acc_v, part_ref.at[pl.ds(wid * 16, 16)])
    wb.wait()


def kernel(z, tok_embeddings, proj_w, proj_b):
    zp = jnp.transpose(z, (0, 2, 3, 1))
    zf = zp.reshape(N, D)

    d, idx, cb_pad = pl.pallas_call(
        _dist_body,
        grid=(NN,),
        in_specs=[
            pl.BlockSpec((D, D), lambda n: (0, 0)),
            pl.BlockSpec((D, D), lambda n: (0, 0)),
            pl.BlockSpec((D, K), lambda n: (0, 0)),
            pl.BlockSpec((D, 1), lambda n: (0, 0)),
            pl.BlockSpec((1, D), lambda n: (0, 0)),
            pl.BlockSpec((NT, D), lambda n: (n, 0)),
        ],
        out_specs=[
            pl.BlockSpec((NT, K), lambda n: (n, 0)),
            pl.BlockSpec((NT,), lambda n: (n,)),
            pl.BlockSpec((K, CP), lambda n: (0, 0)),
        ],
        out_shape=[
            jax.ShapeDtypeStruct((N, K), jnp.float32),
            jax.ShapeDtypeStruct((N,), jnp.int32),
            jax.ShapeDtypeStruct((K, CP), jnp.float32),
        ],
        scratch_shapes=[
            pltpu.VMEM((D, K), jnp.float32),
            pltpu.VMEM((1, K), jnp.float32),
        ],
        compiler_params=pltpu.CompilerParams(
            dimension_semantics=("arbitrary",),
        ),
    )(proj_w, proj_w.T, tok_embeddings.T,
      proj_b.reshape(D, 1), proj_b.reshape(1, D), zf)

    mesh = plsc.VectorSubcoreMesh(core_axis_name="c", subcore_axis_name="s")
    zq_st, partials = pl.kernel(
        _gather_st_body,
        mesh=mesh,
        out_type=[
            jax.ShapeDtypeStruct((N, D), jnp.float32),
            jax.ShapeDtypeStruct((NWORK * 16,), jnp.float32),
        ],
        scratch_types=[
            pltpu.VMEM((RPW,), jnp.int32),
            pltpu.VMEM((RPW, CP), jnp.float32),
            pltpu.VMEM((RPW, D), jnp.float32),
            pltpu.VMEM((RPW, D), jnp.float32),
            pltpu.VMEM((16,), jnp.float32),
            pltpu.SemaphoreType.DMA,
            pltpu.SemaphoreType.DMA,
        ],
    )(cb_pad, idx, zf)

    m = jnp.sum(partials) / (N * D)
    loss = m + 0.33 * m
    z_q_out = jnp.transpose(zq_st.reshape(B, H, W, D), (0, 3, 1, 2))
    return (z_q_out, loss, d, idx)
